# Initial kernel scaffold; baseline (speedup 1.0000x reference)
import jax
import jax.numpy as jnp
from jax.experimental import pallas as pl


def kernel(x, Wr, br, W1, b1, W2, b2):
    raise NotImplementedError("write your pallas kernel here")


# speedup vs baseline: 1.0575x; 1.0575x over previous
import jax
import jax.numpy as jnp
from jax.experimental import pallas as pl
from jax.experimental.pallas import tpu as pltpu

E = 8
K = 2
D = 1024
F = 4096
TM = 512


def _router_body(x_ref, wr_ref, br_ref, comb_ref):
    logits = jnp.dot(x_ref[...], wr_ref[...],
                     preferred_element_type=jnp.float32) + br_ref[...]
    m = jnp.max(logits, axis=-1, keepdims=True)
    eg = jnp.exp(logits - m)
    gates = eg / jnp.sum(eg, axis=-1, keepdims=True)
    v1 = jnp.max(gates, axis=-1, keepdims=True)
    i1 = jnp.argmax(gates, axis=-1)[:, None]
    eiota = jax.lax.broadcasted_iota(jnp.int32, gates.shape, 1)
    masked = jnp.where(eiota == i1, -jnp.inf, gates)
    v2 = jnp.max(masked, axis=-1, keepdims=True)
    i2 = jnp.argmax(masked, axis=-1)[:, None]
    s = v1 + v2
    comb_ref[...] = (jnp.where(eiota == i1, v1 / s, 0.0)
                     + jnp.where(eiota == i2, v2 / s, 0.0))


def _ffn_body(x_ref, w1_ref, b1_ref, w2_ref, b2_ref, comb_ref, out_ref, h_ref):
    e = pl.program_id(1)
    h = jnp.dot(x_ref[...], w1_ref[0], preferred_element_type=jnp.float32)
    h_ref[...] = jnp.maximum(h + b1_ref[0], 0.0).astype(jnp.bfloat16)
    y = jnp.dot(h_ref[...], w2_ref[0],
                preferred_element_type=jnp.float32) + b2_ref[0]
    lane = jax.lax.broadcasted_iota(jnp.int32, comb_ref.shape, 1)
    w = jnp.sum(jnp.where(lane == e, comb_ref[...], 0.0), axis=1,
                keepdims=True)
    contrib = y * w

    @pl.when(e == 0)
    def _():
        out_ref[...] = contrib

    @pl.when(e != 0)
    def _():
        out_ref[...] += contrib


def kernel(x, Wr, br, W1, b1, W2, b2):
    orig_shape = x.shape
    xf = x.reshape(-1, D)
    T = xf.shape[0]

    comb = pl.pallas_call(
        _router_body,
        grid=(T // TM,),
        in_specs=[
            pl.BlockSpec((TM, D), lambda m: (m, 0)),
            pl.BlockSpec((D, E), lambda m: (0, 0)),
            pl.BlockSpec((1, E), lambda m: (0, 0)),
        ],
        out_specs=pl.BlockSpec((TM, E), lambda m: (m, 0)),
        out_shape=jax.ShapeDtypeStruct((T, E), jnp.float32),
    )(xf, Wr, br.reshape(1, E))

    x_bf = xf.astype(jnp.bfloat16)
    W1_bf = W1.astype(jnp.bfloat16)
    W2_bf = W2.astype(jnp.bfloat16)

    out = pl.pallas_call(
        _ffn_body,
        grid=(T // TM, E),
        in_specs=[
            pl.BlockSpec((TM, D), lambda m, e: (m, 0)),
            pl.BlockSpec((1, D, F), lambda m, e: (e, 0, 0)),
            pl.BlockSpec((1, 1, F), lambda m, e: (e, 0, 0)),
            pl.BlockSpec((1, F, D), lambda m, e: (e, 0, 0)),
            pl.BlockSpec((1, 1, D), lambda m, e: (e, 0, 0)),
            pl.BlockSpec((TM, E), lambda m, e: (m, 0)),
        ],
        out_specs=pl.BlockSpec((TM, D), lambda m, e: (m, 0)),
        out_shape=jax.ShapeDtypeStruct((T, D), jnp.float32),
        scratch_shapes=[pltpu.VMEM((TM, F), jnp.bfloat16)],
    )(x_bf, W1_bf, b1.reshape(E, 1, F), W2_bf, b2.reshape(E, 1, D), comb)

    return out.reshape(orig_shape)


# speedup vs baseline: 1.8689x; 1.7673x over previous
import functools

import jax
import jax.numpy as jnp
from jax.experimental import pallas as pl
from jax.experimental.pallas import tpu as pltpu
from jax.experimental.pallas import tpu_sc as plsc

E = 8
K = 2
D = 1024
F = 4096
TM = 512
TMG = 512
GW_DISPATCH = 32
GW_COMBINE = 16


def _router_body(x_ref, wr_ref, br_ref, idx_ref, w_ref):
    x = x_ref[...]
    logits = jnp.dot(x, wr_ref[...], preferred_element_type=jnp.float32)
    logits = logits + br_ref[...]
    m = jnp.max(logits, axis=-1, keepdims=True)
    eg = jnp.exp(logits - m)
    gates = eg / jnp.sum(eg, axis=-1, keepdims=True)
    v1 = jnp.max(gates, axis=-1, keepdims=True)
    i1 = jnp.argmax(gates, axis=-1)[:, None]
    eiota = jax.lax.broadcasted_iota(jnp.int32, gates.shape, 1)
    masked = jnp.where(eiota == i1, -jnp.inf, gates)
    v2 = jnp.max(masked, axis=-1, keepdims=True)
    i2 = jnp.argmax(masked, axis=-1)[:, None]
    s = v1 + v2
    idx_ref[...] = jnp.concatenate([i1, i2], axis=1)
    w_ref[...] = jnp.concatenate([v1 / s, v2 / s], axis=1)


def _ffn_body(se_ref, sm_ref, slo_ref, shi_ref,
              x_ref, w1_ref, b1_ref, w2_ref, b2_ref, ws_ref,
              ys_ref, h_ref):
    s = pl.program_id(0)
    lo = slo_ref[s]
    hi = shi_ref[s]
    h = jnp.dot(x_ref[...].astype(jnp.bfloat16), w1_ref[0],
                preferred_element_type=jnp.float32)
    h_ref[...] = jnp.maximum(h + b1_ref[0], 0.0).astype(jnp.bfloat16)
    y = jnp.dot(h_ref[...], w2_ref[0],
                preferred_element_type=jnp.float32) + b2_ref[0]
    y = y * ws_ref[...]
    rows = jax.lax.broadcasted_iota(jnp.int32, (TMG, 1), 0)
    mask = (rows >= lo) & (rows < hi)
    ys_ref[...] = jnp.where(mask, y, ys_ref[...])


def _add_body(a_ref, b_ref, o_ref):
    o_ref[...] = a_ref[...] + b_ref[...]


_NW = 32


def _sc_gather(data, idx, chunk):
    n = idx.shape[0]
    d = data.shape[1]
    bpw = n // _NW
    mesh = plsc.VectorSubcoreMesh(core_axis_name="c", subcore_axis_name="s")

    @functools.partial(
        pl.kernel,
        out_type=jax.ShapeDtypeStruct((n, d), data.dtype),
        mesh=mesh,
        scratch_types=[
            pltpu.VMEM((bpw,), jnp.int32),
            pltpu.VMEM((chunk, d), data.dtype),
            pltpu.SemaphoreType.DMA,
        ],
    )
    def k(data_hbm, idx_hbm, out_hbm, idx_v, rows_v, sem):
        wid = jax.lax.axis_index("s") * 2 + jax.lax.axis_index("c")
        base = wid * bpw
        pltpu.sync_copy(idx_hbm.at[pl.ds(base, bpw)], idx_v)

        @pl.loop(0, bpw // chunk)
        def _(ci):
            off = ci * chunk
            pltpu.async_copy(data_hbm.at[idx_v.at[pl.ds(off, chunk)]],
                             rows_v, sem).wait()
            pltpu.sync_copy(rows_v, out_hbm.at[pl.ds(base + off, chunk)])

    return k(data, idx)


def kernel(x, Wr, br, W1, b1, W2, b2):
    orig_shape = x.shape
    xf = x.reshape(-1, D)
    T = xf.shape[0]
    P = K * T

    top_idx, top_w = pl.pallas_call(
        _router_body,
        grid=(T // TM,),
        in_specs=[
            pl.BlockSpec((TM, D), lambda m: (m, 0)),
            pl.BlockSpec((D, E), lambda m: (0, 0)),
            pl.BlockSpec((1, E), lambda m: (0, 0)),
        ],
        out_specs=[
            pl.BlockSpec((TM, K), lambda m: (m, 0)),
            pl.BlockSpec((TM, K), lambda m: (m, 0)),
        ],
        out_shape=[
            jax.ShapeDtypeStruct((T, K), jnp.int32),
            jax.ShapeDtypeStruct((T, K), jnp.float32),
        ],
    )(xf, Wr, br.reshape(1, E))

    ef = top_idx.reshape(-1)
    onehot = (ef[:, None] == jnp.arange(E, dtype=jnp.int32)[None, :])
    onehot = onehot.astype(jnp.int32)
    counts = jnp.sum(onehot, axis=0)
    offsets = jnp.concatenate(
        [jnp.zeros((1,), jnp.int32), jnp.cumsum(counts, dtype=jnp.int32)])
    rank = jnp.sum((jnp.cumsum(onehot, axis=0) - onehot) * onehot, axis=1)
    position = offsets[ef] + rank
    pair_sorted = jnp.zeros((P,), jnp.int32).at[position].set(
        jnp.arange(P, dtype=jnp.int32))
    token_sorted = pair_sorted // K
    ws_sorted = top_w.reshape(-1)[pair_sorted].reshape(P, 1)

    NT = P // TMG
    NSLOT = NT + E - 1
    gs = offsets[:-1]
    ge = offsets[1:]
    first = gs // TMG
    last = jnp.where(ge > gs, (ge - 1) // TMG, first - 1)
    gtiles = jnp.maximum(last - first + 1, 0)
    slot_e = jnp.repeat(jnp.arange(E, dtype=jnp.int32), gtiles,
                        total_repeat_length=NSLOT)
    gstart = jnp.concatenate(
        [jnp.zeros((1,), jnp.int32), jnp.cumsum(gtiles, dtype=jnp.int32)])
    sidx = jnp.arange(NSLOT, dtype=jnp.int32)
    valid = sidx < gstart[E]
    slot_m = first[slot_e] + (sidx - gstart[slot_e])
    slot_m = jnp.where(valid, slot_m, NT - 1)
    slot_lo = jnp.where(valid, jnp.maximum(gs[slot_e] - slot_m * TMG, 0), 0)
    slot_hi = jnp.where(valid, jnp.minimum(ge[slot_e] - slot_m * TMG, TMG), 0)

    xs = _sc_gather(xf, token_sorted, 64)

    W1_bf = W1.astype(jnp.bfloat16)
    W2_bf = W2.astype(jnp.bfloat16)
    grid_spec = pltpu.PrefetchScalarGridSpec(
        num_scalar_prefetch=4,
        grid=(NSLOT,),
        in_specs=[
            pl.BlockSpec((TMG, D), lambda s, se, sm, slo, shi: (sm[s], 0)),
            pl.BlockSpec((1, D, F), lambda s, se, sm, slo, shi: (se[s], 0, 0)),
            pl.BlockSpec((1, 1, F), lambda s, se, sm, slo, shi: (se[s], 0, 0)),
            pl.BlockSpec((1, F, D), lambda s, se, sm, slo, shi: (se[s], 0, 0)),
            pl.BlockSpec((1, 1, D), lambda s, se, sm, slo, shi: (se[s], 0, 0)),
            pl.BlockSpec((TMG, 1), lambda s, se, sm, slo, shi: (sm[s], 0)),
        ],
        out_specs=pl.BlockSpec((TMG, D), lambda s, se, sm, slo, shi: (sm[s], 0)),
        scratch_shapes=[pltpu.VMEM((TMG, F), jnp.bfloat16)],
    )
    ys = pl.pallas_call(
        _ffn_body,
        grid_spec=grid_spec,
        out_shape=jax.ShapeDtypeStruct((P, D), jnp.float32),
    )(slot_e, slot_m, slot_lo, slot_hi,
      xs, W1_bf, b1.reshape(E, 1, F), W2_bf, b2.reshape(E, 1, D), ws_sorted)

    gidx = jnp.concatenate([position[0::K], position[1::K]])
    g = _sc_gather(ys, gidx, 64)

    out = pl.pallas_call(
        _add_body,
        grid=(T // TM,),
        in_specs=[
            pl.BlockSpec((TM, D), lambda m: (m, 0)),
            pl.BlockSpec((TM, D), lambda m: (m + T // TM, 0)),
        ],
        out_specs=pl.BlockSpec((TM, D), lambda m: (m, 0)),
        out_shape=jax.ShapeDtypeStruct((T, D), jnp.float32),
    )(g, g)

    return out.reshape(orig_shape)


# speedup vs baseline: 2.0854x; 1.1158x over previous
import functools

import jax
import jax.numpy as jnp
from jax.experimental import pallas as pl
from jax.experimental.pallas import tpu as pltpu
from jax.experimental.pallas import tpu_sc as plsc

E = 8
K = 2
D = 1024
F = 4096
TM = 512
TMG = 512
_NW = 32
_CH = 64


def _router_body(x_ref, wr_ref, br_ref, idx_ref, w_ref):
    x = x_ref[...]
    logits = jnp.dot(x, wr_ref[...], preferred_element_type=jnp.float32)
    logits = logits + br_ref[...]
    m = jnp.max(logits, axis=-1, keepdims=True)
    eg = jnp.exp(logits - m)
    gates = eg / jnp.sum(eg, axis=-1, keepdims=True)
    v1 = jnp.max(gates, axis=-1, keepdims=True)
    i1 = jnp.argmax(gates, axis=-1)[:, None]
    eiota = jax.lax.broadcasted_iota(jnp.int32, gates.shape, 1)
    masked = jnp.where(eiota == i1, -jnp.inf, gates)
    v2 = jnp.max(masked, axis=-1, keepdims=True)
    i2 = jnp.argmax(masked, axis=-1)[:, None]
    s = v1 + v2
    idx_ref[...] = jnp.concatenate([i1, i2], axis=1)
    w_ref[...] = jnp.concatenate([v1 / s, v2 / s], axis=1)


def _ffn_body(se_ref, sm_ref, slo_ref, shi_ref,
              x_ref, w1_ref, b1_ref, w2_ref, b2_ref,
              ys_ref, h_ref):
    s = pl.program_id(0)
    lo = slo_ref[s]
    hi = shi_ref[s]
    h = jnp.dot(x_ref[...].astype(jnp.bfloat16), w1_ref[0],
                preferred_element_type=jnp.float32)
    h_ref[...] = jnp.maximum(h + b1_ref[0], 0.0).astype(jnp.bfloat16)
    y = jnp.dot(h_ref[...], w2_ref[0],
                preferred_element_type=jnp.float32) + b2_ref[0]
    rows = jax.lax.broadcasted_iota(jnp.int32, (TMG, 1), 0)
    mask = (rows >= lo) & (rows < hi)
    ys_ref[...] = jnp.where(mask, y, ys_ref[...])


def _add_body(a_ref, b_ref, w_ref, o_ref):
    w = w_ref[...]
    o_ref[...] = a_ref[...] * w[:, 0:1] + b_ref[...] * w[:, 1:2]


def _sc_dispatch(xf, pos0, pos1):
    T, d = xf.shape
    tpw = T // _NW
    C = tpw // _CH
    mesh = plsc.VectorSubcoreMesh(core_axis_name="c", subcore_axis_name="s")

    @functools.partial(
        pl.kernel,
        out_type=jax.ShapeDtypeStruct((K * T, d), xf.dtype),
        mesh=mesh,
        scratch_types=[
            pltpu.VMEM((C, _CH), jnp.int32),
            pltpu.VMEM((C, _CH), jnp.int32),
            pltpu.VMEM((_CH, d), xf.dtype),
            pltpu.SemaphoreType.DMA,
        ],
    )
    def k(x_hbm, p0_hbm, p1_hbm, out_hbm, i0_v, i1_v, rows_v, sem):
        wid = jax.lax.axis_index("s") * 2 + jax.lax.axis_index("c")
        tb = wid * tpw
        pltpu.sync_copy(p0_hbm.at[wid], i0_v)
        pltpu.sync_copy(p1_hbm.at[wid], i1_v)

        @pl.loop(0, C)
        def _(ci):
            pltpu.sync_copy(x_hbm.at[pl.ds(tb + ci * _CH, _CH)], rows_v)
            pltpu.sync_copy(rows_v, out_hbm.at[i0_v.at[ci]])
            pltpu.sync_copy(rows_v, out_hbm.at[i1_v.at[ci]])

    return k(xf, pos0, pos1)


def _sc_gather(data, idx):
    n = idx.shape[0]
    d = data.shape[1]
    bpw = n // _NW
    mesh = plsc.VectorSubcoreMesh(core_axis_name="c", subcore_axis_name="s")

    @functools.partial(
        pl.kernel,
        out_type=jax.ShapeDtypeStruct((n, d), data.dtype),
        mesh=mesh,
        scratch_types=[
            pltpu.VMEM((bpw,), jnp.int32),
            pltpu.VMEM((_CH, d), data.dtype),
            pltpu.SemaphoreType.DMA,
        ],
    )
    def k(data_hbm, idx_hbm, out_hbm, idx_v, rows_v, sem):
        wid = jax.lax.axis_index("s") * 2 + jax.lax.axis_index("c")
        base = wid * bpw
        pltpu.sync_copy(idx_hbm.at[pl.ds(base, bpw)], idx_v)

        @pl.loop(0, bpw // _CH)
        def _(ci):
            off = ci * _CH
            pltpu.async_copy(data_hbm.at[idx_v.at[pl.ds(off, _CH)]],
                             rows_v, sem).wait()
            pltpu.sync_copy(rows_v, out_hbm.at[pl.ds(base + off, _CH)])

    return k(data, idx)


def kernel(x, Wr, br, W1, b1, W2, b2):
    orig_shape = x.shape
    xf = x.reshape(-1, D)
    T = xf.shape[0]
    P = K * T

    top_idx, top_w = pl.pallas_call(
        _router_body,
        grid=(T // TM,),
        in_specs=[
            pl.BlockSpec((TM, D), lambda m: (m, 0)),
            pl.BlockSpec((D, E), lambda m: (0, 0)),
            pl.BlockSpec((1, E), lambda m: (0, 0)),
        ],
        out_specs=[
            pl.BlockSpec((TM, K), lambda m: (m, 0)),
            pl.BlockSpec((TM, K), lambda m: (m, 0)),
        ],
        out_shape=[
            jax.ShapeDtypeStruct((T, K), jnp.int32),
            jax.ShapeDtypeStruct((T, K), jnp.float32),
        ],
    )(xf, Wr, br.reshape(1, E))

    ef = top_idx.reshape(-1)
    onehot = (ef[:, None] == jnp.arange(E, dtype=jnp.int32)[None, :])
    onehot = onehot.astype(jnp.int32)
    counts = jnp.sum(onehot, axis=0)
    offsets = jnp.concatenate(
        [jnp.zeros((1,), jnp.int32), jnp.cumsum(counts, dtype=jnp.int32)])
    rank = jnp.sum((jnp.cumsum(onehot, axis=0) - onehot) * onehot, axis=1)
    position = offsets[ef] + rank
    pos2 = position.reshape(T, K)
    pos0 = pos2[:, 0]
    pos1 = pos2[:, 1]

    NT = P // TMG
    NSLOT = NT + E - 1
    gs = offsets[:-1]
    ge = offsets[1:]
    first = gs // TMG
    last = jnp.where(ge > gs, (ge - 1) // TMG, first - 1)
    gtiles = jnp.maximum(last - first + 1, 0)
    slot_e = jnp.repeat(jnp.arange(E, dtype=jnp.int32), gtiles,
                        total_repeat_length=NSLOT)
    gstart = jnp.concatenate(
        [jnp.zeros((1,), jnp.int32), jnp.cumsum(gtiles, dtype=jnp.int32)])
    sidx = jnp.arange(NSLOT, dtype=jnp.int32)
    valid = sidx < gstart[E]
    slot_m = first[slot_e] + (sidx - gstart[slot_e])
    slot_m = jnp.where(valid, slot_m, NT - 1)
    slot_lo = jnp.where(valid, jnp.maximum(gs[slot_e] - slot_m * TMG, 0), 0)
    slot_hi = jnp.where(valid, jnp.minimum(ge[slot_e] - slot_m * TMG, TMG), 0)

    tpw = T // _NW
    xs = _sc_dispatch(xf,
                      pos0.reshape(_NW, tpw // _CH, _CH),
                      pos1.reshape(_NW, tpw // _CH, _CH))

    W1_bf = W1.astype(jnp.bfloat16)
    W2_bf = W2.astype(jnp.bfloat16)
    grid_spec = pltpu.PrefetchScalarGridSpec(
        num_scalar_prefetch=4,
        grid=(NSLOT,),
        in_specs=[
            pl.BlockSpec((TMG, D), lambda s, se, sm, slo, shi: (sm[s], 0)),
            pl.BlockSpec((1, D, F), lambda s, se, sm, slo, shi: (se[s], 0, 0)),
            pl.BlockSpec((1, 1, F), lambda s, se, sm, slo, shi: (se[s], 0, 0)),
            pl.BlockSpec((1, F, D), lambda s, se, sm, slo, shi: (se[s], 0, 0)),
            pl.BlockSpec((1, 1, D), lambda s, se, sm, slo, shi: (se[s], 0, 0)),
        ],
        out_specs=pl.BlockSpec((TMG, D), lambda s, se, sm, slo, shi: (sm[s], 0)),
        scratch_shapes=[pltpu.VMEM((TMG, F), jnp.bfloat16)],
    )
    ys = pl.pallas_call(
        _ffn_body,
        grid_spec=grid_spec,
        out_shape=jax.ShapeDtypeStruct((P, D), jnp.float32),
    )(slot_e, slot_m, slot_lo, slot_hi,
      xs, W1_bf, b1.reshape(E, 1, F), W2_bf, b2.reshape(E, 1, D))

    gidx = jnp.concatenate([pos0, pos1])
    g = _sc_gather(ys, gidx)

    out = pl.pallas_call(
        _add_body,
        grid=(T // TM,),
        in_specs=[
            pl.BlockSpec((TM, D), lambda m: (m, 0)),
            pl.BlockSpec((TM, D), lambda m: (m + T // TM, 0)),
            pl.BlockSpec((TM, K), lambda m: (m, 0)),
        ],
        out_specs=pl.BlockSpec((TM, D), lambda m: (m, 0)),
        out_shape=jax.ShapeDtypeStruct((T, D), jnp.float32),
    )(g, g, top_w)

    return out.reshape(orig_shape)
